# baseline (speedup 1.0000x reference)
import jax
import jax.numpy as jnp
from jax.experimental import pallas as pl
from jax.experimental.pallas import tpu as pltpu

_A = 8400
_AP = 8448
_NP = 64
_NC = 80
_TOPK = 9
_EPS = 1e-9
_L0 = 6400
_L1 = 1600
_INF = 3.0e38
_BIGI = 1 << 30
_BPS = 2


def _top9_exact(d, width):
    cidx = jax.lax.broadcasted_iota(jnp.int32, (_NP, width), 1)
    s = jnp.zeros((_NP, width), jnp.float32)
    for _ in range(_TOPK):
        m = jnp.min(d, axis=1, keepdims=True)
        j = jnp.min(jnp.where(d == m, cidx, _BIGI), axis=1, keepdims=True)
        pick = cidx == j
        s = s + pick.astype(jnp.float32)
        d = jnp.where(pick, _INF, d)
    return s


def _atss_body(anc_ref, gt_ref, lab_ref, box_ref, sco_ref):
    for u in range(_BPS):
        _one_batch(anc_ref, gt_ref, lab_ref, box_ref, sco_ref, u)


def _one_batch(anc_ref, gt_ref, lab_ref, box_ref, sco_ref, u):
    a = anc_ref[:, :]
    ax0 = a[0:1, :]
    ay0 = a[1:2, :]
    ax1 = a[2:3, :]
    ay1 = a[3:4, :]
    g = gt_ref[u]
    gx0 = g[:, 0:1]
    gy0 = g[:, 1:2]
    gx1 = g[:, 2:3]
    gy1 = g[:, 3:4]
    glab = g[:, 4:5]
    gmask = g[:, 5:6]

    acx = (ax0 + ax1) * 0.5
    acy = (ay0 + ay1) * 0.5
    aarea = (ax1 - ax0) * (ay1 - ay0)
    gcx = (gx0 + gx1) * 0.5
    gcy = (gy0 + gy1) * 0.5
    garea = (gx1 - gx0) * (gy1 - gy0)

    dx = gcx - acx
    dy = gcy - acy
    aidx = jax.lax.broadcasted_iota(jnp.int32, (1, _AP), 1)
    dist = jnp.where(aidx < _A, jnp.sqrt(dx * dx + dy * dy), _INF)

    dlo = dist[:, 0:_L0]
    dhi = dist[:, _L0:_AP]
    cidx = jax.lax.broadcasted_iota(jnp.int32, (_NP, _AP - _L0), 1)
    d1 = jnp.where(cidx < _L1, dhi, _INF)
    d2 = jnp.where(cidx >= _L1, dhi, _INF)
    sel = jnp.concatenate(
        [_top9_exact(dlo, _L0),
         _top9_exact(d1, _AP - _L0) + _top9_exact(d2, _AP - _L0)],
        axis=1)

    inter = (jnp.maximum(jnp.minimum(gx1, ax1) - jnp.maximum(gx0, ax0), 0.0)
             * jnp.maximum(jnp.minimum(gy1, ay1) - jnp.maximum(gy0, ay0), 0.0))
    iou = inter / (garea + aarea - inter + _EPS)

    selm = sel * gmask
    iou_c = iou * selm
    mean = jnp.sum(iou_c, axis=1, keepdims=True) * (1.0 / (3 * _TOPK))
    dvar = iou_c - mean
    var = jnp.sum(sel * dvar * dvar, axis=1, keepdims=True) * (1.0 / (3 * _TOPK - 1))
    thr = mean + jnp.sqrt(jnp.maximum(var, 0.0))
    topk_f = jnp.where(iou_c > thr, selm, jnp.zeros_like(selm))

    m_in = jnp.minimum(jnp.minimum(acx - gx0, acy - gy0),
                       jnp.minimum(gx1 - acx, gy1 - acy))
    maskp = topk_f * (m_in > _EPS).astype(jnp.float32) * gmask

    colsum = jnp.sum(maskp, axis=0, keepdims=True)
    multi = colsum > 1.0
    gidx = jax.lax.broadcasted_iota(jnp.int32, (_NP, _AP), 0)
    miou = jnp.max(iou, axis=0, keepdims=True)
    firstmax = jnp.min(jnp.where(iou == miou, gidx, _BIGI), axis=0,
                       keepdims=True)
    ismax = (gidx == firstmax).astype(jnp.float32)
    maskp2 = jnp.where(multi, ismax, maskp)

    possum = jnp.sum(maskp2, axis=0, keepdims=True)
    pos = possum > 0.0
    firstpos = jnp.min(jnp.where(maskp2 > 0.0, gidx, _BIGI), axis=0,
                       keepdims=True)
    assigned = jnp.where(pos, firstpos, jnp.zeros_like(firstpos))

    onehot = (gidx == assigned).astype(jnp.float32)
    labi = jnp.sum(onehot * glab, axis=0, keepdims=True).astype(jnp.int32)
    labi = jnp.where(pos, labi, jnp.full_like(labi, -1))
    lab_ref[u] = labi

    rows = [jax.lax.bitcast_convert_type(labi, jnp.float32)]
    for j in range(4):
        rows.append(jnp.sum(onehot * g[:, j:j + 1], axis=0, keepdims=True))
    rows.append(jnp.zeros((3, _AP), jnp.float32))
    tr = jnp.transpose(jnp.concatenate(rows, axis=0), (1, 0))
    box_ref[u] = tr[0:_A, 1:5]
    lab_c = jax.lax.bitcast_convert_type(tr[0:_A, 0:1], jnp.int32)
    cls = jax.lax.broadcasted_iota(jnp.int32, (_A, _NC), 1)
    sco_ref[u] = jnp.where(lab_c == cls, jnp.float32(1.0), jnp.float32(0.0))


def kernel(anchor_bboxes, num_anchors_list, gt_labels, gt_bboxes, pad_gt_mask,
           bg_index):
    B, n, _ = gt_bboxes.shape
    anc = jnp.zeros((8, _AP), jnp.float32).at[:4, :_A].set(
        anchor_bboxes.astype(jnp.float32).T)
    packed = jnp.concatenate(
        [gt_bboxes.astype(jnp.float32),
         gt_labels.astype(jnp.float32),
         pad_gt_mask.astype(jnp.float32),
         jnp.zeros((B, n, 2), jnp.float32)], axis=2)
    packed = jnp.pad(packed, ((0, 0), (0, _NP - n), (0, 0)))

    cparams = pltpu.CompilerParams(dimension_semantics=("parallel",))
    lab, box, sco = pl.pallas_call(
        _atss_body,
        grid=(B // _BPS,),
        in_specs=[
            pl.BlockSpec((8, _AP), lambda b: (0, 0)),
            pl.BlockSpec((_BPS, _NP, 8), lambda b: (b, 0, 0)),
        ],
        out_specs=[
            pl.BlockSpec((_BPS, 1, _AP), lambda b: (b, 0, 0)),
            pl.BlockSpec((_BPS, _A, 4), lambda b: (b, 0, 0)),
            pl.BlockSpec((_BPS, _A, _NC), lambda b: (b, 0, 0)),
        ],
        out_shape=[
            jax.ShapeDtypeStruct((B, 1, _AP), jnp.int32),
            jax.ShapeDtypeStruct((B, _A, 4), jnp.float32),
            jax.ShapeDtypeStruct((B, _A, _NC), jnp.float32),
        ],
        compiler_params=cparams,
    )(anc, packed)

    labels = lab[:, 0, :_A]
    labels = jnp.where(labels < 0, bg_index, labels).astype(jnp.int32)
    return labels, box, sco


# speedup vs baseline: 1.4569x; 1.4569x over previous
import jax
import jax.numpy as jnp
from jax.experimental import pallas as pl
from jax.experimental.pallas import tpu as pltpu

_A = 8400
_AP = 8448
_NP = 64
_NC = 80
_TOPK = 9
_EPS = 1e-9
_L0 = 6400
_L1 = 1600
_INF = 3.0e38
_BIGI = 1 << 30
_BPS = 1


def _top9_knockout(d, width):
    cidx = jax.lax.broadcasted_iota(jnp.int32, (_NP, width), 1)
    for _ in range(_TOPK):
        m = jnp.min(d, axis=1, keepdims=True)
        j = jnp.min(jnp.where(d == m, cidx, _BIGI), axis=1, keepdims=True)
        d = jnp.where(cidx == j, _INF, d)
    return d


def _atss_body(anc_ref, gt_ref, lab_ref, box_ref, sco_ref):
    for u in range(_BPS):
        _one_batch(anc_ref, gt_ref, lab_ref, box_ref, sco_ref, u)


def _one_batch(anc_ref, gt_ref, lab_ref, box_ref, sco_ref, u):
    a = anc_ref[:, :]
    ax0 = a[0:1, :]
    ay0 = a[1:2, :]
    ax1 = a[2:3, :]
    ay1 = a[3:4, :]
    g = gt_ref[u]
    gx0 = g[:, 0:1]
    gy0 = g[:, 1:2]
    gx1 = g[:, 2:3]
    gy1 = g[:, 3:4]
    glab = g[:, 4:5]
    gmask = g[:, 5:6]

    acx = (ax0 + ax1) * 0.5
    acy = (ay0 + ay1) * 0.5
    aarea = (ax1 - ax0) * (ay1 - ay0)
    gcx = (gx0 + gx1) * 0.5
    gcy = (gy0 + gy1) * 0.5
    garea = (gx1 - gx0) * (gy1 - gy0)

    dx = gcx - acx
    dy = gcy - acy
    dist = jnp.sqrt(dx * dx + dy * dy)

    dlo = dist[:, 0:_L0]
    dhi = dist[:, _L0:_AP]
    cidx = jax.lax.broadcasted_iota(jnp.int32, (_NP, _AP - _L0), 1)
    d1 = jnp.where(cidx < _L1, dhi, _INF)
    d2 = jnp.where(cidx >= _L1, dhi, _INF)
    k0 = _top9_knockout(dlo, _L0)
    k1 = _top9_knockout(d1, _AP - _L0)
    k2 = _top9_knockout(d2, _AP - _L0)
    sel_hi = jnp.where(cidx < _L1, (k1 == _INF).astype(jnp.float32),
                       (k2 == _INF).astype(jnp.float32))
    sel = jnp.concatenate(
        [(k0 == _INF).astype(jnp.float32), sel_hi], axis=1)

    inter = (jnp.maximum(jnp.minimum(gx1, ax1) - jnp.maximum(gx0, ax0), 0.0)
             * jnp.maximum(jnp.minimum(gy1, ay1) - jnp.maximum(gy0, ay0), 0.0))
    iou = inter / (garea + aarea - inter + _EPS)

    selm = sel * gmask
    iou_c = iou * selm
    mean = jnp.sum(iou_c, axis=1, keepdims=True) * (1.0 / (3 * _TOPK))
    dvar = iou_c - mean
    var = jnp.sum(sel * dvar * dvar, axis=1, keepdims=True) * (1.0 / (3 * _TOPK - 1))
    thr = mean + jnp.sqrt(jnp.maximum(var, 0.0))
    topk_f = jnp.where(iou_c > thr, selm, jnp.zeros_like(selm))

    m_in = jnp.minimum(jnp.minimum(acx - gx0, acy - gy0),
                       jnp.minimum(gx1 - acx, gy1 - acy))
    maskp = topk_f * (m_in > _EPS).astype(jnp.float32) * gmask

    colsum = jnp.sum(maskp, axis=0, keepdims=True)
    multi = colsum > 1.0
    gidx = jax.lax.broadcasted_iota(jnp.int32, (_NP, _AP), 0)
    miou = jnp.max(iou, axis=0, keepdims=True)
    firstmax = jnp.min(jnp.where(iou == miou, gidx, _BIGI), axis=0,
                       keepdims=True)
    ismax = (gidx == firstmax).astype(jnp.float32)
    maskp2 = jnp.where(multi, ismax, maskp)

    possum = jnp.sum(maskp2, axis=0, keepdims=True)
    pos = possum > 0.0
    firstpos = jnp.min(jnp.where(maskp2 > 0.0, gidx, _BIGI), axis=0,
                       keepdims=True)
    assigned = jnp.where(pos, firstpos, jnp.zeros_like(firstpos))

    onehot = (gidx == assigned).astype(jnp.float32)
    labi = jnp.sum(onehot * glab, axis=0, keepdims=True).astype(jnp.int32)
    labi = jnp.where(pos, labi, jnp.full_like(labi, -1))
    lab_ref[u] = labi

    rows = [jax.lax.bitcast_convert_type(labi, jnp.float32)]
    for j in range(4):
        rows.append(jnp.sum(onehot * g[:, j:j + 1], axis=0, keepdims=True))
    rows.append(jnp.zeros((3, _AP), jnp.float32))
    tr = jnp.transpose(jnp.concatenate(rows, axis=0), (1, 0))
    box_ref[u] = tr[0:_A, 1:5]
    lab_c = jax.lax.bitcast_convert_type(tr[0:_A, 0:1], jnp.int32)
    cls = jax.lax.broadcasted_iota(jnp.int32, (_A, _NC), 1)
    sco_ref[u] = jnp.where(lab_c == cls, jnp.float32(1.0), jnp.float32(0.0))


def kernel(anchor_bboxes, num_anchors_list, gt_labels, gt_bboxes, pad_gt_mask,
           bg_index):
    B, n, _ = gt_bboxes.shape
    anc = jnp.full((8, _AP), 1e18, jnp.float32).at[:4, :_A].set(
        anchor_bboxes.astype(jnp.float32).T)
    packed = jnp.concatenate(
        [gt_bboxes.astype(jnp.float32),
         gt_labels.astype(jnp.float32),
         pad_gt_mask.astype(jnp.float32),
         jnp.zeros((B, n, 2), jnp.float32)], axis=2)
    packed = jnp.pad(packed, ((0, 0), (0, _NP - n), (0, 0)))

    cparams = pltpu.CompilerParams(dimension_semantics=("parallel",))
    lab, box, sco = pl.pallas_call(
        _atss_body,
        grid=(B // _BPS,),
        in_specs=[
            pl.BlockSpec((8, _AP), lambda b: (0, 0)),
            pl.BlockSpec((_BPS, _NP, 8), lambda b: (b, 0, 0)),
        ],
        out_specs=[
            pl.BlockSpec((_BPS, 1, _AP), lambda b: (b, 0, 0)),
            pl.BlockSpec((_BPS, _A, 4), lambda b: (b, 0, 0)),
            pl.BlockSpec((_BPS, _A, _NC), lambda b: (b, 0, 0)),
        ],
        out_shape=[
            jax.ShapeDtypeStruct((B, 1, _AP), jnp.int32),
            jax.ShapeDtypeStruct((B, _A, 4), jnp.float32),
            jax.ShapeDtypeStruct((B, _A, _NC), jnp.float32),
        ],
        compiler_params=cparams,
    )(anc, packed)

    labels = lab[:, 0, :_A]
    labels = jnp.where(labels < 0, bg_index, labels).astype(jnp.int32)
    return labels, box, sco


# speedup vs baseline: 1.5655x; 1.0746x over previous
import jax
import jax.numpy as jnp
from jax.experimental import pallas as pl
from jax.experimental.pallas import tpu as pltpu

_A = 8400
_AP = 8448
_NP = 56
_NC = 80
_TOPK = 9
_EPS = 1e-9
_L0 = 6400
_L1 = 1600
_INF = 3.0e38
_BIGI = 1 << 30
_BPS = 1


def _top9_knockout(d, width):
    cidx = jax.lax.broadcasted_iota(jnp.int32, (_NP, width), 1)
    for _ in range(_TOPK):
        m = jnp.min(d, axis=1, keepdims=True)
        j = jnp.min(jnp.where(d == m, cidx, _BIGI), axis=1, keepdims=True)
        d = jnp.where(cidx == j, _INF, d)
    return d


def _atss_body(anc_ref, gt_ref, lab_ref, box_ref, sco_ref):
    for u in range(_BPS):
        _one_batch(anc_ref, gt_ref, lab_ref, box_ref, sco_ref, u)


def _one_batch(anc_ref, gt_ref, lab_ref, box_ref, sco_ref, u):
    a = anc_ref[:, :]
    ax0 = a[0:1, :]
    ay0 = a[1:2, :]
    ax1 = a[2:3, :]
    ay1 = a[3:4, :]
    g = gt_ref[u]
    gx0 = g[:, 0:1]
    gy0 = g[:, 1:2]
    gx1 = g[:, 2:3]
    gy1 = g[:, 3:4]
    glab = g[:, 4:5]
    gmask = g[:, 5:6]

    acx = (ax0 + ax1) * 0.5
    acy = (ay0 + ay1) * 0.5
    aarea = (ax1 - ax0) * (ay1 - ay0)
    gcx = (gx0 + gx1) * 0.5
    gcy = (gy0 + gy1) * 0.5
    garea = (gx1 - gx0) * (gy1 - gy0)

    dx = gcx - acx
    dy = gcy - acy
    dist = jnp.sqrt(dx * dx + dy * dy)

    dlo = dist[:, 0:_L0]
    dhi = dist[:, _L0:_AP]
    cidx = jax.lax.broadcasted_iota(jnp.int32, (_NP, _AP - _L0), 1)
    d1 = jnp.where(cidx < _L1, dhi, _INF)
    d2 = jnp.where(cidx >= _L1, dhi, _INF)
    k0 = _top9_knockout(dlo, _L0)
    k1 = _top9_knockout(d1, _AP - _L0)
    k2 = _top9_knockout(d2, _AP - _L0)
    sel_hi = jnp.where(cidx < _L1, (k1 == _INF).astype(jnp.float32),
                       (k2 == _INF).astype(jnp.float32))
    sel = jnp.concatenate(
        [(k0 == _INF).astype(jnp.float32), sel_hi], axis=1)

    inter = (jnp.maximum(jnp.minimum(gx1, ax1) - jnp.maximum(gx0, ax0), 0.0)
             * jnp.maximum(jnp.minimum(gy1, ay1) - jnp.maximum(gy0, ay0), 0.0))
    iou = inter / (garea + aarea - inter + _EPS)

    selm = sel * gmask
    iou_c = iou * selm
    mean = jnp.sum(iou_c, axis=1, keepdims=True) * (1.0 / (3 * _TOPK))
    dvar = iou_c - mean
    var = jnp.sum(sel * dvar * dvar, axis=1, keepdims=True) * (1.0 / (3 * _TOPK - 1))
    thr = mean + jnp.sqrt(jnp.maximum(var, 0.0))
    topk_f = jnp.where(iou_c > thr, selm, jnp.zeros_like(selm))

    m_in = jnp.minimum(jnp.minimum(acx - gx0, acy - gy0),
                       jnp.minimum(gx1 - acx, gy1 - acy))
    maskp = topk_f * (m_in > _EPS).astype(jnp.float32) * gmask

    colsum = jnp.sum(maskp, axis=0, keepdims=True)
    multi = colsum > 1.0
    gidx = jax.lax.broadcasted_iota(jnp.int32, (_NP, _AP), 0)
    miou = jnp.max(iou, axis=0, keepdims=True)
    firstmax = jnp.min(jnp.where(iou == miou, gidx, _BIGI), axis=0,
                       keepdims=True)
    ismax = (gidx == firstmax).astype(jnp.float32)
    maskp2 = jnp.where(multi, ismax, maskp)

    possum = jnp.sum(maskp2, axis=0, keepdims=True)
    pos = possum > 0.0
    firstpos = jnp.min(jnp.where(maskp2 > 0.0, gidx, _BIGI), axis=0,
                       keepdims=True)
    assigned = jnp.where(pos, firstpos, jnp.zeros_like(firstpos))

    onehot = (gidx == assigned).astype(jnp.float32)
    labi = jnp.sum(onehot * glab, axis=0, keepdims=True).astype(jnp.int32)
    labi = jnp.where(pos, labi, jnp.full_like(labi, -1))
    lab_ref[u] = labi

    rows = [jax.lax.bitcast_convert_type(labi, jnp.float32)]
    for j in range(4):
        rows.append(jnp.sum(onehot * g[:, j:j + 1], axis=0, keepdims=True))
    rows.append(jnp.zeros((3, _AP), jnp.float32))
    tr = jnp.transpose(jnp.concatenate(rows, axis=0), (1, 0))
    box_ref[u] = tr[0:_A, 1:5]
    lab_c = jax.lax.bitcast_convert_type(tr[0:_A, 0:1], jnp.int32)
    cls = jax.lax.broadcasted_iota(jnp.int32, (_A, _NC), 1)
    sco_ref[u] = jnp.where(lab_c == cls, jnp.float32(1.0), jnp.float32(0.0))


def kernel(anchor_bboxes, num_anchors_list, gt_labels, gt_bboxes, pad_gt_mask,
           bg_index):
    B, n, _ = gt_bboxes.shape
    anc = jnp.full((8, _AP), 1e18, jnp.float32).at[:4, :_A].set(
        anchor_bboxes.astype(jnp.float32).T)
    packed = jnp.concatenate(
        [gt_bboxes.astype(jnp.float32),
         gt_labels.astype(jnp.float32),
         pad_gt_mask.astype(jnp.float32),
         jnp.zeros((B, n, 2), jnp.float32)], axis=2)
    packed = jnp.pad(packed, ((0, 0), (0, _NP - n), (0, 0)))

    cparams = pltpu.CompilerParams(dimension_semantics=("parallel",))
    lab, box, sco = pl.pallas_call(
        _atss_body,
        grid=(B // _BPS,),
        in_specs=[
            pl.BlockSpec((8, _AP), lambda b: (0, 0)),
            pl.BlockSpec((_BPS, _NP, 8), lambda b: (b, 0, 0)),
        ],
        out_specs=[
            pl.BlockSpec((_BPS, 1, _AP), lambda b: (b, 0, 0)),
            pl.BlockSpec((_BPS, _A, 4), lambda b: (b, 0, 0)),
            pl.BlockSpec((_BPS, _A, _NC), lambda b: (b, 0, 0)),
        ],
        out_shape=[
            jax.ShapeDtypeStruct((B, 1, _AP), jnp.int32),
            jax.ShapeDtypeStruct((B, _A, 4), jnp.float32),
            jax.ShapeDtypeStruct((B, _A, _NC), jnp.float32),
        ],
        compiler_params=cparams,
    )(anc, packed)

    labels = lab[:, 0, :_A]
    labels = jnp.where(labels < 0, bg_index, labels).astype(jnp.int32)
    return labels, box, sco


# speedup vs baseline: 1.5765x; 1.0070x over previous
import jax
import jax.numpy as jnp
from jax.experimental import pallas as pl
from jax.experimental.pallas import tpu as pltpu

_A = 8400
_AP = 8448
_NP = 56
_NC = 80
_TOPK = 9
_EPS = 1e-9
_L0 = 6400
_L1 = 1600
_INF = 3.0e38
_BIGI = 1 << 30
_BPS = 1


def _top9_knockout(d, width):
    cidx = jax.lax.broadcasted_iota(jnp.int32, (_NP, width), 1)
    for _ in range(_TOPK):
        m = jnp.min(d, axis=1, keepdims=True)
        j = jnp.min(jnp.where(d == m, cidx, _BIGI), axis=1, keepdims=True)
        d = jnp.where(cidx == j, _INF, d)
    return d


def _atss_body(anc_ref, gt_ref, lab_ref, box_ref, sco_ref):
    for u in range(_BPS):
        _one_batch(anc_ref, gt_ref, lab_ref, box_ref, sco_ref, u)


def _one_batch(anc_ref, gt_ref, lab_ref, box_ref, sco_ref, u):
    a = anc_ref[:, :]
    ax0 = a[0:1, :]
    ay0 = a[1:2, :]
    ax1 = a[2:3, :]
    ay1 = a[3:4, :]
    g = gt_ref[u]
    gx0 = g[:, 0:1]
    gy0 = g[:, 1:2]
    gx1 = g[:, 2:3]
    gy1 = g[:, 3:4]
    glab = g[:, 4:5]
    gmask = g[:, 5:6]

    acx = (ax0 + ax1) * 0.5
    acy = (ay0 + ay1) * 0.5
    aarea = (ax1 - ax0) * (ay1 - ay0)
    gcx = (gx0 + gx1) * 0.5
    gcy = (gy0 + gy1) * 0.5
    garea = (gx1 - gx0) * (gy1 - gy0)

    dx = gcx - acx
    dy = gcy - acy
    dist = jnp.sqrt(dx * dx + dy * dy)

    dlo = dist[:, 0:_L0]
    dhi = dist[:, _L0:_AP]
    cidx = jax.lax.broadcasted_iota(jnp.int32, (_NP, _AP - _L0), 1)
    d1 = jnp.where(cidx < _L1, dhi, _INF)
    d2 = jnp.where(cidx >= _L1, dhi, _INF)
    k0 = _top9_knockout(dlo, _L0)
    k1 = _top9_knockout(d1, _AP - _L0)
    k2 = _top9_knockout(d2, _AP - _L0)
    kfull = jnp.concatenate(
        [k0, jnp.where(cidx < _L1, k1, k2)], axis=1)

    inter = (jnp.maximum(jnp.minimum(gx1, ax1) - jnp.maximum(gx0, ax0), 0.0)
             * jnp.maximum(jnp.minimum(gy1, ay1) - jnp.maximum(gy0, ay0), 0.0))
    iou = inter / (garea + aarea - inter + _EPS)

    zero = jnp.zeros((_NP, _AP), jnp.float32)
    selb = kfull == _INF
    iou_c = jnp.where(selb, iou * gmask, zero)
    mean = jnp.sum(iou_c, axis=1, keepdims=True) * (1.0 / (3 * _TOPK))
    dvar = iou_c - mean
    var = jnp.sum(jnp.where(selb, dvar * dvar, zero), axis=1,
                  keepdims=True) * (1.0 / (3 * _TOPK - 1))
    thr = mean + jnp.sqrt(jnp.maximum(var, 0.0))
    topk_f = jnp.where(iou_c > thr, jnp.broadcast_to(gmask, (_NP, _AP)), zero)

    m_in = jnp.minimum(jnp.minimum(acx - gx0, acy - gy0),
                       jnp.minimum(gx1 - acx, gy1 - acy))
    maskp = topk_f * (m_in > _EPS).astype(jnp.float32) * gmask

    colsum = jnp.sum(maskp, axis=0, keepdims=True)
    multi = colsum > 1.0
    gidx = jax.lax.broadcasted_iota(jnp.int32, (_NP, _AP), 0)
    miou = jnp.max(iou, axis=0, keepdims=True)
    firstmax = jnp.min(jnp.where(iou == miou, gidx, _BIGI), axis=0,
                       keepdims=True)
    ismax = (gidx == firstmax).astype(jnp.float32)
    maskp2 = jnp.where(multi, ismax, maskp)

    possum = jnp.sum(maskp2, axis=0, keepdims=True)
    pos = possum > 0.0
    firstpos = jnp.min(jnp.where(maskp2 > 0.0, gidx, _BIGI), axis=0,
                       keepdims=True)
    assigned = jnp.where(pos, firstpos, jnp.zeros_like(firstpos))

    onehot = (gidx == assigned).astype(jnp.float32)
    labi = jnp.sum(onehot * glab, axis=0, keepdims=True).astype(jnp.int32)
    labi = jnp.where(pos, labi, jnp.full_like(labi, -1))
    lab_ref[u] = labi

    rows = [jax.lax.bitcast_convert_type(labi, jnp.float32)]
    for j in range(4):
        rows.append(jnp.sum(onehot * g[:, j:j + 1], axis=0, keepdims=True))
    rows.append(jnp.zeros((3, _AP), jnp.float32))
    tr = jnp.transpose(jnp.concatenate(rows, axis=0), (1, 0))
    box_ref[u] = tr[0:_A, 1:5]
    lab_c = jax.lax.bitcast_convert_type(tr[0:_A, 0:1], jnp.int32)
    cls = jax.lax.broadcasted_iota(jnp.int32, (_A, _NC), 1)
    sco_ref[u] = jnp.where(lab_c == cls, jnp.float32(1.0), jnp.float32(0.0))


def kernel(anchor_bboxes, num_anchors_list, gt_labels, gt_bboxes, pad_gt_mask,
           bg_index):
    B, n, _ = gt_bboxes.shape
    anc = jnp.full((8, _AP), 1e18, jnp.float32).at[:4, :_A].set(
        anchor_bboxes.astype(jnp.float32).T)
    packed = jnp.concatenate(
        [gt_bboxes.astype(jnp.float32),
         gt_labels.astype(jnp.float32),
         pad_gt_mask.astype(jnp.float32),
         jnp.zeros((B, n, 2), jnp.float32)], axis=2)
    packed = jnp.pad(packed, ((0, 0), (0, _NP - n), (0, 0)))

    cparams = pltpu.CompilerParams(dimension_semantics=("parallel",))
    lab, box, sco = pl.pallas_call(
        _atss_body,
        grid=(B // _BPS,),
        in_specs=[
            pl.BlockSpec((8, _AP), lambda b: (0, 0)),
            pl.BlockSpec((_BPS, _NP, 8), lambda b: (b, 0, 0)),
        ],
        out_specs=[
            pl.BlockSpec((_BPS, 1, _AP), lambda b: (b, 0, 0)),
            pl.BlockSpec((_BPS, _A, 4), lambda b: (b, 0, 0)),
            pl.BlockSpec((_BPS, _A, _NC), lambda b: (b, 0, 0)),
        ],
        out_shape=[
            jax.ShapeDtypeStruct((B, 1, _AP), jnp.int32),
            jax.ShapeDtypeStruct((B, _A, 4), jnp.float32),
            jax.ShapeDtypeStruct((B, _A, _NC), jnp.float32),
        ],
        compiler_params=cparams,
    )(anc, packed)

    labels = lab[:, 0, :_A]
    labels = jnp.where(labels < 0, bg_index, labels).astype(jnp.int32)
    return labels, box, sco


# speedup vs baseline: 1.5779x; 1.0009x over previous
import jax
import jax.numpy as jnp
from jax.experimental import pallas as pl
from jax.experimental.pallas import tpu as pltpu

_A = 8400
_AP = 8448
_NP = 56
_NC = 80
_TOPK = 9
_EPS = 1e-9
_L0 = 6400
_L1 = 1600
_INF = 3.0e38
_BIGI = 1 << 30
_BPS = 1


def _top9_knockout(d, width):
    cidx = jax.lax.broadcasted_iota(jnp.int32, (_NP, width), 1)
    for _ in range(_TOPK):
        m = jnp.min(d, axis=1, keepdims=True)
        j = jnp.min(jnp.where(d == m, cidx, _BIGI), axis=1, keepdims=True)
        d = jnp.where(cidx == j, _INF, d)
    return d


def _atss_body(anc_ref, gt_ref, lab_ref, box_ref, sco_ref):
    for u in range(_BPS):
        _one_batch(anc_ref, gt_ref, lab_ref, box_ref, sco_ref, u)


def _one_batch(anc_ref, gt_ref, lab_ref, box_ref, sco_ref, u):
    a = anc_ref[:, :]
    ax0 = a[0:1, :]
    ay0 = a[1:2, :]
    ax1 = a[2:3, :]
    ay1 = a[3:4, :]
    g = gt_ref[u]
    gx0 = g[:, 0:1]
    gy0 = g[:, 1:2]
    gx1 = g[:, 2:3]
    gy1 = g[:, 3:4]
    glab = g[:, 4:5]
    gmask = g[:, 5:6]

    acx = (ax0 + ax1) * 0.5
    acy = (ay0 + ay1) * 0.5
    aarea = (ax1 - ax0) * (ay1 - ay0)
    gcx = (gx0 + gx1) * 0.5
    gcy = (gy0 + gy1) * 0.5
    garea = (gx1 - gx0) * (gy1 - gy0)

    dx = gcx - acx
    dy = gcy - acy
    dist = jnp.sqrt(dx * dx + dy * dy)

    dlo = dist[:, 0:_L0]
    dhi = dist[:, _L0:_AP]
    cidx = jax.lax.broadcasted_iota(jnp.int32, (_NP, _AP - _L0), 1)
    c1 = cidx < _L1
    d1 = jnp.where(c1, dhi, _INF)
    d2 = jnp.where(c1, _INF, dhi)
    k0 = _top9_knockout(dlo, _L0)
    k1 = _top9_knockout(d1, _AP - _L0)
    k2 = _top9_knockout(d2, _AP - _L0)
    kfull = jnp.concatenate(
        [k0, jnp.where(c1, k1, k2)], axis=1)

    inter = (jnp.maximum(jnp.minimum(gx1, ax1) - jnp.maximum(gx0, ax0), 0.0)
             * jnp.maximum(jnp.minimum(gy1, ay1) - jnp.maximum(gy0, ay0), 0.0))
    iou = inter / (garea + aarea - inter + _EPS)

    zero = jnp.zeros((_NP, _AP), jnp.float32)
    selb = kfull == _INF
    iou_c = jnp.where(selb, iou * gmask, zero)
    mean = jnp.sum(iou_c, axis=1, keepdims=True) * (1.0 / (3 * _TOPK))
    dvar = iou_c - mean
    var = jnp.sum(jnp.where(selb, dvar * dvar, zero), axis=1,
                  keepdims=True) * (1.0 / (3 * _TOPK - 1))
    thr = mean + jnp.sqrt(jnp.maximum(var, 0.0))
    topk_f = jnp.where(iou_c > thr, jnp.broadcast_to(gmask, (_NP, _AP)), zero)

    m_in = jnp.minimum(jnp.minimum(acx - gx0, acy - gy0),
                       jnp.minimum(gx1 - acx, gy1 - acy))
    maskp = topk_f * (m_in > _EPS).astype(jnp.float32) * gmask

    colsum = jnp.sum(maskp, axis=0, keepdims=True)
    multi = colsum > 1.0
    gidx = jax.lax.broadcasted_iota(jnp.int32, (_NP, _AP), 0)
    miou = jnp.max(iou, axis=0, keepdims=True)
    firstmax = jnp.min(jnp.where(iou == miou, gidx, _BIGI), axis=0,
                       keepdims=True)
    ismax = (gidx == firstmax).astype(jnp.float32)
    maskp2 = jnp.where(multi, ismax, maskp)

    possum = jnp.sum(maskp2, axis=0, keepdims=True)
    pos = possum > 0.0
    firstpos = jnp.min(jnp.where(maskp2 > 0.0, gidx, _BIGI), axis=0,
                       keepdims=True)
    assigned = jnp.where(pos, firstpos, jnp.zeros_like(firstpos))

    onehot = (gidx == assigned).astype(jnp.float32)
    labi = jnp.sum(onehot * glab, axis=0, keepdims=True).astype(jnp.int32)
    labi = jnp.where(pos, labi, jnp.full_like(labi, -1))
    lab_ref[u] = labi

    rows = [jax.lax.bitcast_convert_type(labi, jnp.float32)]
    for j in range(4):
        rows.append(jnp.sum(onehot * g[:, j:j + 1], axis=0, keepdims=True))
    rows.append(jnp.zeros((3, _AP), jnp.float32))
    tr = jnp.transpose(jnp.concatenate(rows, axis=0), (1, 0))
    box_ref[u] = tr[0:_A, 1:5]
    lab_c = jax.lax.bitcast_convert_type(tr[0:_A, 0:1], jnp.int32)
    cls = jax.lax.broadcasted_iota(jnp.int32, (_A, _NC), 1)
    sco_ref[u] = jnp.where(lab_c == cls, jnp.float32(1.0), jnp.float32(0.0))


def kernel(anchor_bboxes, num_anchors_list, gt_labels, gt_bboxes, pad_gt_mask,
           bg_index):
    B, n, _ = gt_bboxes.shape
    anc = jnp.full((8, _AP), 1e18, jnp.float32).at[:4, :_A].set(
        anchor_bboxes.astype(jnp.float32).T)
    packed = jnp.concatenate(
        [gt_bboxes.astype(jnp.float32),
         gt_labels.astype(jnp.float32),
         pad_gt_mask.astype(jnp.float32),
         jnp.zeros((B, n, 2), jnp.float32)], axis=2)
    packed = jnp.pad(packed, ((0, 0), (0, _NP - n), (0, 0)))

    cparams = pltpu.CompilerParams(dimension_semantics=("parallel",))
    lab, box, sco = pl.pallas_call(
        _atss_body,
        grid=(B // _BPS,),
        in_specs=[
            pl.BlockSpec((8, _AP), lambda b: (0, 0)),
            pl.BlockSpec((_BPS, _NP, 8), lambda b: (b, 0, 0)),
        ],
        out_specs=[
            pl.BlockSpec((_BPS, 1, _AP), lambda b: (b, 0, 0)),
            pl.BlockSpec((_BPS, _A, 4), lambda b: (b, 0, 0)),
            pl.BlockSpec((_BPS, _A, _NC), lambda b: (b, 0, 0)),
        ],
        out_shape=[
            jax.ShapeDtypeStruct((B, 1, _AP), jnp.int32),
            jax.ShapeDtypeStruct((B, _A, 4), jnp.float32),
            jax.ShapeDtypeStruct((B, _A, _NC), jnp.float32),
        ],
        compiler_params=cparams,
    )(anc, packed)

    labels = lab[:, 0, :_A]
    labels = jnp.where(labels < 0, bg_index, labels).astype(jnp.int32)
    return labels, box, sco
